# baseline (device time: 53918 ns/iter reference)
import jax
import jax.numpy as jnp
from jax import lax
from jax.experimental import pallas as pl
from jax.experimental.pallas import tpu as pltpu

N_DEV = 4
N_RING = 4


def kernel(x, dy):
    k_per, m = x.shape
    k_per2, n = dy.shape
    assert k_per == k_per2
    m_out = m // N_DEV
    nq = n // N_RING

    def body(x_ref, dy_ref, out_ref, dy_bf, stage, comm,
             send_sems, recv_sems):
        my = lax.axis_index("i")
        left = (my + N_DEV - 1) % N_DEV
        right = (my + 1) % N_DEV

        def pchunk(c, ring, dy_src):
            xs = x_ref[:, pl.ds(c * m_out, m_out)].astype(jnp.bfloat16)
            dys = dy_src[:, ring * nq:(ring + 1) * nq]
            if dys.dtype != jnp.bfloat16:
                dys = dys.astype(jnp.bfloat16)
            return lax.dot_general(
                xs, dys,
                dimension_numbers=(((0,), (0,)), ((), ())),
                preferred_element_type=jnp.float32,
            )

        def c_send0(ring):
            if ring < 2:
                return (my + N_DEV - 1) % N_DEV
            return (my + 1) % N_DEV

        def c_recv(ring, s):
            if ring < 2:
                return (my + 2 * N_DEV - 2 - s) % N_DEV
            return (my + 2 + s) % N_DEV

        def make_rdma(ring, s, src):
            return pltpu.make_async_remote_copy(
                src_ref=src,
                dst_ref=comm.at[ring, s],
                send_sem=send_sems.at[ring, s],
                recv_sem=recv_sems.at[ring, s],
                device_id=(right if ring < 2 else left,),
                device_id_type=pl.DeviceIdType.MESH,
            )

        stage[0, :, :] = pchunk(c_send0(0), 0, dy_ref[...]).astype(jnp.bfloat16)
        stage[2, :, :] = pchunk(c_send0(2), 2, dy_ref[...]).astype(jnp.bfloat16)

        barrier_sem = pltpu.get_barrier_semaphore()
        for nbr in (left, right):
            pl.semaphore_signal(
                barrier_sem, inc=1,
                device_id=(nbr,), device_id_type=pl.DeviceIdType.MESH,
            )
        pl.semaphore_wait(barrier_sem, 2)

        rdmas = [None] * N_RING
        for ring in (0, 2):
            rdmas[ring] = make_rdma(ring, 0, stage.at[ring])
            rdmas[ring].start()

        for ring in (1, 3):
            stage[ring, :, :] = pchunk(
                c_send0(ring), ring, dy_ref[...]).astype(jnp.bfloat16)
            rdmas[ring] = make_rdma(ring, 0, stage.at[ring])
            rdmas[ring].start()

        half = 2 * nq
        dy_bf[:, :half] = dy_ref[:, :half].astype(jnp.bfloat16)
        p = [None] * N_RING
        p[0] = pchunk(c_recv(0, 0), 0, dy_bf[...])
        dy_bf[:, half:] = dy_ref[:, half:].astype(jnp.bfloat16)
        p[2] = pchunk(c_recv(2, 0), 2, dy_bf[...])

        for s in range(N_DEV - 1):
            for ring in (0, 2, 1, 3):
                if s == 0 and ring == 1:
                    p[1] = pchunk(c_recv(1, 0), 1, dy_bf[...])
                    p[3] = pchunk(c_recv(3, 0), 3, dy_bf[...])
                rdmas[ring].wait()
                if s < N_DEV - 2:
                    comm[ring, s, :, :] = (
                        comm[ring, s, :, :].astype(jnp.float32) + p[ring]
                    ).astype(jnp.bfloat16)
                    rdmas[ring] = make_rdma(ring, s + 1, comm.at[ring, s])
                    rdmas[ring].start()
                else:
                    out_ref[:, ring * nq:(ring + 1) * nq] = (
                        comm[ring, s, :, :].astype(jnp.float32) + p[ring]
                    )
            if s < N_DEV - 2:
                for ring in (0, 2, 1, 3):
                    p[ring] = pchunk(c_recv(ring, s + 1), ring, dy_bf[...])

    return pl.pallas_call(
        body,
        out_shape=jax.ShapeDtypeStruct((m_out, n), jnp.float32),
        in_specs=[
            pl.BlockSpec(memory_space=pltpu.VMEM),
            pl.BlockSpec(memory_space=pltpu.VMEM),
        ],
        out_specs=pl.BlockSpec(memory_space=pltpu.VMEM),
        scratch_shapes=[
            pltpu.VMEM((k_per, n), jnp.bfloat16),
            pltpu.VMEM((N_RING, m_out, nq), jnp.bfloat16),
            pltpu.VMEM((N_RING, N_DEV - 1, m_out, nq), jnp.bfloat16),
            pltpu.SemaphoreType.DMA((N_RING, N_DEV - 1)),
            pltpu.SemaphoreType.DMA((N_RING, N_DEV - 1)),
        ],
        compiler_params=pltpu.CompilerParams(
            collective_id=0,
            vmem_limit_bytes=100 * 1024 * 1024,
        ),
    )(x, dy)


# device time: 50838 ns/iter; 1.0606x vs baseline; 1.0606x over previous
import jax
import jax.numpy as jnp
from jax import lax
from jax.experimental import pallas as pl
from jax.experimental.pallas import tpu as pltpu

N_DEV = 4
N_RING = 4
_ORDER = (0, 2, 1, 3)


def kernel(x, dy):
    k_per, m = x.shape
    k_per2, n = dy.shape
    assert k_per == k_per2
    m_out = m // N_DEV
    nq = n // N_RING
    nr2 = N_RING // 2

    def body(x_ref, dy_hbm, out_ref, dy_bf, stg, stage, comm,
             copy_sems, send_sems, recv_sems):
        my = lax.axis_index("i")
        left = (my + N_DEV - 1) % N_DEV
        right = (my + 1) % N_DEV

        def strip_copy(i, slot):
            ring = _ORDER[i]
            return pltpu.make_async_copy(
                dy_hbm.at[:, ring * nq:(ring + 1) * nq],
                stg.at[slot],
                copy_sems.at[i],
            )

        def pchunk(c, ring):
            xs = x_ref[:, pl.ds(c * m_out, m_out)].astype(jnp.bfloat16)
            dys = dy_bf[:, ring * nq:(ring + 1) * nq]
            return lax.dot_general(
                xs, dys,
                dimension_numbers=(((0,), (0,)), ((), ())),
                preferred_element_type=jnp.float32,
            )

        def c_send0(ring):
            return (my + N_DEV - 1) % N_DEV if ring < nr2 else (my + 1) % N_DEV

        def c_recv(ring, s):
            if ring < nr2:
                return (my + 2 * N_DEV - 2 - s) % N_DEV
            return (my + 2 + s) % N_DEV

        def make_rdma(ring, s, src):
            return pltpu.make_async_remote_copy(
                src_ref=src,
                dst_ref=comm.at[ring, s],
                send_sem=send_sems.at[ring, s],
                recv_sem=recv_sems.at[ring, s],
                device_id=(right if ring < nr2 else left,),
                device_id_type=pl.DeviceIdType.MESH,
            )

        copies = [strip_copy(0, 0), strip_copy(1, 1)]
        copies[0].start()
        copies[1].start()

        barrier_sem = pltpu.get_barrier_semaphore()
        for nbr in (left, right):
            pl.semaphore_signal(
                barrier_sem, inc=1,
                device_id=(nbr,), device_id_type=pl.DeviceIdType.MESH,
            )
        pl.semaphore_wait(barrier_sem, 2)

        rdmas = [None] * N_RING
        for i, ring in enumerate(_ORDER):
            copies[i].wait()
            if i + 2 < N_RING:
                copies.append(strip_copy(i + 2, i % 2))
                copies[i + 2].start()
            dy_bf[:, ring * nq:(ring + 1) * nq] = stg[i % 2].astype(jnp.bfloat16)
            stage[ring, :, :] = pchunk(c_send0(ring), ring).astype(jnp.bfloat16)
            rdmas[ring] = make_rdma(ring, 0, stage.at[ring])
            rdmas[ring].start()

        p = [None] * N_RING
        for ring in _ORDER:
            p[ring] = pchunk(c_recv(ring, 0), ring)

        for s in range(N_DEV - 1):
            for ring in _ORDER:
                rdmas[ring].wait()
                if s < N_DEV - 2:
                    comm[ring, s, :, :] = (
                        comm[ring, s, :, :].astype(jnp.float32) + p[ring]
                    ).astype(jnp.bfloat16)
                    rdmas[ring] = make_rdma(ring, s + 1, comm.at[ring, s])
                    rdmas[ring].start()
                else:
                    out_ref[:, ring * nq:(ring + 1) * nq] = (
                        comm[ring, s, :, :].astype(jnp.float32) + p[ring]
                    )
            if s < N_DEV - 2:
                for ring in _ORDER:
                    p[ring] = pchunk(c_recv(ring, s + 1), ring)

    return pl.pallas_call(
        body,
        out_shape=jax.ShapeDtypeStruct((m_out, n), jnp.float32),
        in_specs=[
            pl.BlockSpec(memory_space=pltpu.VMEM),
            pl.BlockSpec(memory_space=pl.ANY),
        ],
        out_specs=pl.BlockSpec(memory_space=pltpu.VMEM),
        scratch_shapes=[
            pltpu.VMEM((k_per, n), jnp.bfloat16),
            pltpu.VMEM((2, k_per, n // N_RING), jnp.float32),
            pltpu.VMEM((N_RING, m_out, n // N_RING), jnp.bfloat16),
            pltpu.VMEM((N_RING, N_DEV - 1, m_out, n // N_RING), jnp.bfloat16),
            pltpu.SemaphoreType.DMA((N_RING,)),
            pltpu.SemaphoreType.DMA((N_RING, N_DEV - 1)),
            pltpu.SemaphoreType.DMA((N_RING, N_DEV - 1)),
        ],
        compiler_params=pltpu.CompilerParams(
            collective_id=0,
            vmem_limit_bytes=100 * 1024 * 1024,
        ),
    )(x, dy)
